# Initial kernel scaffold; baseline (speedup 1.0000x reference)
import jax
import jax.numpy as jnp
from jax.experimental import pallas as pl


def kernel(x, table):
    raise NotImplementedError("write your pallas kernel here")


# speedup vs baseline: 2.4944x; 2.4944x over previous
import jax
import jax.numpy as jnp
from jax import lax
from jax.experimental import pallas as pl
from jax.experimental.pallas import tpu as pltpu
from jax.experimental.pallas import tpu_sc as plsc

BATCH = 16384
HIST = 200
D = 64
LANES = 16
NCOL = D // LANES

NW = 32
EPW = BATCH // NW
G = 4
NG = EPW // G
IDX_PER_G = G * HIST
CHUNK = 80
NCHUNK = IDX_PER_G // CHUNK
OUT_BUF = 64
GPF = OUT_BUF // G


def _body(x_ref, table_ref, out_ref, idx_v, rows_v, out_v, sem):
    nc = 2
    wid = lax.axis_index("s") * nc + lax.axis_index("c")
    base_elem = wid * EPW
    scale = jnp.full((LANES,), 1.0 / HIST, dtype=jnp.float32)

    def group_body(g, carry):
        gbase = base_elem + g * G
        pltpu.sync_copy(x_ref.at[pl.ds(gbase * HIST, IDX_PER_G)], idx_v)
        copies = []
        for k in range(NCHUNK):
            copies.append(pltpu.async_copy(
                table_ref.at[idx_v.at[pl.ds(k * CHUNK, CHUNK)]],
                rows_v.at[pl.ds(k * CHUNK, CHUNK), :],
                sem))
        for c in copies:
            c.wait()

        orow0 = (g % GPF) * G
        for e in range(G):
            rb = e * HIST

            def red_body(j, accs):
                r0 = rb + j * 8
                new = list(accs)
                for u in range(8):
                    for c in range(NCOL):
                        new[c] = new[c] + rows_v[r0 + u, pl.ds(c * LANES, LANES)]
                return tuple(new)

            z = jnp.zeros((LANES,), jnp.float32)
            accs = lax.fori_loop(0, HIST // 8, red_body, (z,) * NCOL)
            for c in range(NCOL):
                out_v[orow0 + e, pl.ds(c * LANES, LANES)] = accs[c] * scale

        @pl.when(g % GPF == GPF - 1)
        def _flush():
            ob = base_elem + (g // GPF) * OUT_BUF
            pltpu.sync_copy(out_v, out_ref.at[pl.ds(ob, OUT_BUF), :])

        return carry

    lax.fori_loop(0, NG, group_body, 0)


def kernel(x, table):
    xf = x.reshape(-1).astype(jnp.int32)
    mesh = plsc.VectorSubcoreMesh(core_axis_name="c", subcore_axis_name="s")
    f = pl.kernel(
        _body,
        out_type=jax.ShapeDtypeStruct((BATCH, D), jnp.float32),
        mesh=mesh,
        scratch_types=[
            pltpu.VMEM((IDX_PER_G,), jnp.int32),
            pltpu.VMEM((IDX_PER_G, D), jnp.float32),
            pltpu.VMEM((OUT_BUF, D), jnp.float32),
            pltpu.SemaphoreType.DMA,
        ],
        compiler_params=pltpu.CompilerParams(use_tc_tiling_on_sc=False),
    )
    return f(xf, table)


# speedup vs baseline: 3.2350x; 1.2969x over previous
import jax
import jax.numpy as jnp
from jax import lax
from jax.experimental import pallas as pl
from jax.experimental.pallas import tpu as pltpu
from jax.experimental.pallas import tpu_sc as plsc

BATCH = 16384
HIST = 200
D = 64
LANES = 16
NCOL = D // LANES

NW = 32
EPW = BATCH // NW
G = 4
NG = EPW // G
NGP = NG // 2
IDX_PER_G = G * HIST
CHUNK = 80
NCHUNK = IDX_PER_G // CHUNK
OUT_BUF = 64
GPF = OUT_BUF // G


def _body(x_ref, table_ref, out_ref, idx_a, idx_b, rows_a, rows_b, out_v,
          sem_a, sem_b):
    nc = 2
    wid = lax.axis_index("s") * nc + lax.axis_index("c")
    base_elem = wid * EPW
    scale = jnp.full((LANES,), 1.0 / HIST, dtype=jnp.float32)

    def stage_idx(g, idx_v):
        pltpu.sync_copy(
            x_ref.at[pl.ds((base_elem + g * G) * HIST, IDX_PER_G)], idx_v)

    def fire(idx_v, rows_v, sem):
        for k in range(NCHUNK):
            pltpu.async_copy(
                table_ref.at[idx_v.at[pl.ds(k * CHUNK, CHUNK)]],
                rows_v.at[pl.ds(k * CHUNK, CHUNK), :],
                sem)

    def drain(idx_v, rows_v, sem):
        for k in range(NCHUNK):
            pltpu.make_async_copy(
                table_ref.at[idx_v.at[pl.ds(k * CHUNK, CHUNK)]],
                rows_v.at[pl.ds(k * CHUNK, CHUNK), :],
                sem).wait()

    def reduce(g, rows_v):
        orow0 = (g % GPF) * G
        for e in range(G):
            rb = e * HIST

            def red_body(j, accs):
                r0 = rb + j * 8
                new = list(accs)
                for u in range(8):
                    for c in range(NCOL):
                        new[c] = new[c] + rows_v[r0 + u, pl.ds(c * LANES, LANES)]
                return tuple(new)

            z = jnp.zeros((LANES,), jnp.float32)
            accs = lax.fori_loop(0, HIST // 8, red_body, (z,) * NCOL)
            for c in range(NCOL):
                out_v[orow0 + e, pl.ds(c * LANES, LANES)] = accs[c] * scale

    stage_idx(0, idx_a)
    fire(idx_a, rows_a, sem_a)

    def pair_body(i, carry):
        g0 = 2 * i
        g1 = 2 * i + 1

        stage_idx(g1, idx_b)
        fire(idx_b, rows_b, sem_b)

        drain(idx_a, rows_a, sem_a)
        reduce(g0, rows_a)

        @pl.when(i < NGP - 1)
        def _refire_a():
            stage_idx(g0 + 2, idx_a)
            fire(idx_a, rows_a, sem_a)

        drain(idx_b, rows_b, sem_b)
        reduce(g1, rows_b)

        @pl.when(i % (GPF // 2) == GPF // 2 - 1)
        def _flush():
            ob = base_elem + (g1 // GPF) * OUT_BUF
            pltpu.sync_copy(out_v, out_ref.at[pl.ds(ob, OUT_BUF), :])

        return carry

    lax.fori_loop(0, NGP, pair_body, 0)


def kernel(x, table):
    xf = x.reshape(-1).astype(jnp.int32)
    mesh = plsc.VectorSubcoreMesh(core_axis_name="c", subcore_axis_name="s")
    f = pl.kernel(
        _body,
        out_type=jax.ShapeDtypeStruct((BATCH, D), jnp.float32),
        mesh=mesh,
        scratch_types=[
            pltpu.VMEM((IDX_PER_G,), jnp.int32),
            pltpu.VMEM((IDX_PER_G,), jnp.int32),
            pltpu.VMEM((IDX_PER_G, D), jnp.float32),
            pltpu.VMEM((IDX_PER_G, D), jnp.float32),
            pltpu.VMEM((OUT_BUF, D), jnp.float32),
            pltpu.SemaphoreType.DMA,
            pltpu.SemaphoreType.DMA,
        ],
        compiler_params=pltpu.CompilerParams(use_tc_tiling_on_sc=False),
    )
    return f(xf, table)
